# Initial kernel scaffold; baseline (speedup 1.0000x reference)
import jax
import jax.numpy as jnp
from jax.experimental import pallas as pl


def kernel(x, label):
    raise NotImplementedError("write your pallas kernel here")


# speedup vs baseline: 28.7552x; 28.7552x over previous
import functools

import numpy as np
import jax
import jax.numpy as jnp
from jax import lax
from jax.experimental import pallas as pl
from jax.experimental.pallas import tpu as pltpu
from jax.experimental.pallas import tpu_sc as plsc

_B = 8
_A = 3
_F = 64
_NCH = 85
_NL = 16
_NCLS = 80
_CHP = 88
_NTILES = 32
_PER_TILE = _B * _NL * _CHP // _NTILES

_AW = (np.array([13.0, 28.0, 62.0], np.float32) / 4096.0).astype(np.float32)
_AH = (np.array([16.0, 32.0, 35.0], np.float32) / 4096.0).astype(np.float32)
_ATAN_A = np.arctan(_AW / _AH).astype(np.float32)


def _fit_atan_coeffs(deg=10, n=2000):
    k = np.arange(n)
    u = (np.cos(np.pi * (k + 0.5) / n) + 1.0) / 2.0
    x = np.sqrt(u)
    f = np.where(x == 0, 1.0, np.arctan(x) / np.where(x == 0, 1.0, x))
    vand = np.vander(u, deg + 1, increasing=True)
    c, *_ = np.linalg.lstsq(vand, f, rcond=None)
    return [float(ci) for ci in c]


_ATAN_C = _fit_atan_coeffs()


def _atan_pos(z):
    inv = z > 1.0
    y = jnp.where(inv, 1.0 / jnp.where(inv, z, 1.0), z)
    u = y * y
    p = jnp.full_like(u, _ATAN_C[-1])
    for c in _ATAN_C[-2::-1]:
        p = p * u + c
    at = y * p
    return jnp.where(inv, float(np.pi / 2) - at, at)


def _bce(p, t):
    p_safe = jnp.where(p > 0, p, 1.0)
    logp = jnp.where(p > 0, jnp.maximum(jnp.log(p_safe), -100.0), -100.0)
    q = 1.0 - p
    q_safe = jnp.where(q > 0, q, 1.0)
    logq = jnp.where(q > 0, jnp.maximum(jnp.log(q_safe), -100.0), -100.0)
    return -(t * logp + (1.0 - t) * logq)


def _label_prep(label):
    def body(lab_ref, metaf_ref, metai_ref, idx_ref):
        lab = lab_ref[...]
        lx = (lab[:, :, 0] + lab[:, :, 2]) / 16.0
        ly = (lab[:, :, 1] + lab[:, :, 3]) / 16.0
        lw = lab[:, :, 2] / 8.0
        lh = lab[:, :, 3] / 8.0
        cc = lab[:, :, 4].astype(jnp.int32)
        ii = lx.astype(jnp.int32)
        jj = ly.astype(jnp.int32)
        area_a = lw * lh
        at1 = _atan_pos(lw / jnp.where(lh == 0.0, 1e-16, lh))
        best = jnp.zeros(lw.shape, jnp.int32)
        bestv = jnp.full(lw.shape, -jnp.inf, jnp.float32)
        for k in range(_A):
            aw = _AW[k]
            ah = _AH[k]
            iw = jnp.minimum(lw, aw)
            ih = jnp.minimum(lh, ah)
            en = ((iw > 0.0) & (ih > 0.0)).astype(jnp.float32)
            area_i = iw * ih * en
            union = area_a + aw * ah - area_i
            iou = area_i / jnp.where(union == 0.0, 1e-16, union)
            c2 = jnp.maximum(lw, aw) ** 2 + jnp.maximum(lh, ah) ** 2 + 1e-16
            rho2 = ((lw - aw) ** 2 + (lh - ah) ** 2) / 4.0
            v = (4.0 / np.pi ** 2) * (_ATAN_A[k] - at1) ** 2
            denom = 1.0 - iou + v
            alpha = v / jnp.where(denom == 0.0, 1e-16, denom)
            ciou = iou - rho2 / c2 - alpha * v
            take = ciou > bestv
            best = jnp.where(take, k, best)
            bestv = jnp.where(take, ciou, bestv)
        aw_b = jnp.where(best == 0, _AW[0], jnp.where(best == 1, _AW[1], _AW[2]))
        ah_b = jnp.where(best == 0, _AH[0], jnp.where(best == 1, _AH[1], _AH[2]))
        t0 = lx - ii.astype(jnp.float32)
        sc = jnp.sqrt(2.0 - area_a / float(_F * _F))
        lwt = jnp.log(lw / aw_b + 1e-16)
        lht = jnp.log(lh / ah_b + 1e-16)
        metaf_ref[:, 0, :] = lx
        metaf_ref[:, 1, :] = ly
        metaf_ref[:, 2, :] = lw
        metaf_ref[:, 3, :] = lh
        metaf_ref[:, 4, :] = sc
        metaf_ref[:, 5, :] = lwt
        metaf_ref[:, 6, :] = lht
        metaf_ref[:, 7, :] = t0
        metai_ref[:, 0, :] = ii
        metai_ref[:, 1, :] = jj
        metai_ref[:, 2, :] = best
        metai_ref[:, 3, :] = cc
        bidx = lax.broadcasted_iota(jnp.int32, (_B, _NL), 0)
        base = ((bidx * _A + best) * _NCH) * (_F * _F) + jj * _F + ii
        chv = jnp.minimum(
            lax.broadcasted_iota(jnp.int32, (_B, _NL, _CHP), 2), _NCH - 1
        ) * (_F * _F)
        idx_ref[...] = base[:, :, None] + chv

    return pl.pallas_call(
        body,
        out_shape=[
            jax.ShapeDtypeStruct((_B, 8, _NL), jnp.float32),
            jax.ShapeDtypeStruct((_B, 4, _NL), jnp.int32),
            jax.ShapeDtypeStruct((_B, _NL, _CHP), jnp.int32),
        ],
    )(label)


def _gather_cells(xflat, idx2):
    mesh = plsc.VectorSubcoreMesh(core_axis_name="c", subcore_axis_name="s")

    @functools.partial(
        pl.kernel,
        out_type=jax.ShapeDtypeStruct((_NTILES, _PER_TILE), jnp.float32),
        mesh=mesh,
        scratch_types=[
            pltpu.VMEM((_PER_TILE,), jnp.int32),
            pltpu.VMEM((_PER_TILE,), jnp.float32),
            pltpu.SemaphoreType.DMA,
        ],
    )
    def gk(x_hbm, idx_hbm, out_hbm, idx_v, val_v, sem):
        wid = lax.axis_index("s") * 2 + lax.axis_index("c")
        pltpu.sync_copy(idx_hbm.at[wid], idx_v)
        copies = []
        for q in range(_PER_TILE // 88):
            sl = pl.ds(q * 88, 88)
            copies.append(pltpu.async_copy(x_hbm.at[idx_v.at[sl]], val_v.at[sl], sem))
        for cp in copies:
            cp.wait()
        pltpu.sync_copy(val_v, out_hbm.at[wid])

    return gk(xflat, idx2)


def _dense_pass(xr, metaf, metai):
    def body(x_ref, mf_ref, mi_ref, out_ref):
        b = pl.program_id(0)

        @pl.when(b == 0)
        def _init():
            out_ref[...] = jnp.zeros((1, 128), jnp.float32)

        xb = x_ref[0]
        mf = mf_ref[0]
        mi = mi_ref[0]
        lane = lax.broadcasted_iota(jnp.int32, (_A, _F * _F), 1)
        gi = lane % _F
        gj = lane // _F
        anc = lax.broadcasted_iota(jnp.int32, (_A, _F * _F), 0)
        anc3 = lax.broadcasted_iota(jnp.int32, (_A, 1), 0)
        awv = jnp.where(anc3 == 0, float(_AW[0]),
                        jnp.where(anc3 == 1, float(_AW[1]), float(_AW[2])))
        ahv = jnp.where(anc3 == 0, float(_AH[0]),
                        jnp.where(anc3 == 1, float(_AH[1]), float(_AH[2])))
        s0 = jax.nn.sigmoid(xb[:, 0, :])
        s1 = jax.nn.sigmoid(xb[:, 1, :])
        px = s0 + gi.astype(jnp.float32)
        py = s1 + gj.astype(jnp.float32)
        pw = jnp.exp(xb[:, 2, :] * awv)
        ph = jnp.exp(xb[:, 3, :] * ahv)
        s4 = jax.nn.sigmoid(xb[:, 4, :])
        area_a = (pw - px) * (ph - py)
        best = jnp.zeros((_A, _F * _F), jnp.float32)
        ist = jnp.zeros((_A, _F * _F), jnp.bool_)
        for l in range(_NL):
            lx = mf[0, l]
            ly = mf[1, l]
            lw = mf[2, l]
            lh = mf[3, l]
            tlx = jnp.maximum(px, lx)
            tly = jnp.maximum(py, ly)
            brx = jnp.minimum(pw, lw)
            bry = jnp.minimum(ph, lh)
            en = ((tlx < brx) & (tly < bry)).astype(jnp.float32)
            ai = (brx - tlx) * (bry - tly) * en
            area_b = (lw - lx) * (lh - ly)
            union = area_a + area_b - ai
            iou = ai / jnp.where(union == 0.0, 1e-16, union)
            best = jnp.maximum(best, iou)
            ist = ist | ((anc == mi[2, l]) & (gj == mi[1, l]) & (gi == mi[0, l]))
        istf = ist.astype(jnp.float32)
        objm = jnp.where(ist, 1.0, 1.0 - (best > 0.5).astype(jnp.float32))
        p = s4 * objm
        lobj = jnp.sum(_bce(p, istf))
        l2 = jnp.sum((p - istf) ** 2)
        lane128 = lax.broadcasted_iota(jnp.int32, (1, 128), 1)
        acc = (jnp.where(lane128 == 0, lobj, 0.0)
               + jnp.where(lane128 == 1, l2, 0.0))
        out_ref[...] += acc

    return pl.pallas_call(
        body,
        grid=(_B,),
        in_specs=[
            pl.BlockSpec((1, _A, 8, _F * _F), lambda b: (b, 0, 0, 0)),
            pl.BlockSpec((1, 8, _NL), lambda b: (b, 0, 0)),
            pl.BlockSpec((1, 4, _NL), lambda b: (b, 0, 0)),
        ],
        out_specs=pl.BlockSpec((1, 128), lambda b: (0, 0)),
        out_shape=jax.ShapeDtypeStruct((1, 128), jnp.float32),
    )(xr, metaf, metai)


def _corrections(cellv, metaf, metai):
    def body(cv_ref, mf_ref, mi_ref, out_ref):
        cv = cv_ref[...]
        sc = mf_ref[:, 4, :]
        lwt = mf_ref[:, 5, :]
        lht = mf_ref[:, 6, :]
        t0 = mf_ref[:, 7, :]
        ii = mi_ref[:, 0, :]
        jj = mi_ref[:, 1, :]
        aa = mi_ref[:, 2, :]
        cc = mi_ref[:, 3, :]
        cellid = (aa * _F + jj) * _F + ii
        same = cellid[:, :, None] == cellid[:, None, :]
        li = lax.broadcasted_iota(jnp.int32, (_B, _NL, _NL), 1)
        ui = lax.broadcasted_iota(jnp.int32, (_B, _NL, _NL), 2)
        is_last = ~jnp.any(same & (ui > li), axis=2)
        samef = same.astype(jnp.float32)
        conehot = (
            lax.broadcasted_iota(jnp.int32, (_B, _NL, _NCLS), 2) == cc[:, :, None]
        ).astype(jnp.float32)
        uni = lax.dot_general(
            samef, conehot, (((2,), (1,)), ((0,), (0,))),
            preferred_element_type=jnp.float32,
        )
        unif = (uni > 0.0).astype(jnp.float32)
        s0 = jax.nn.sigmoid(cv[:, :, 0])
        s1 = jax.nn.sigmoid(cv[:, :, 1])
        v2 = cv[:, :, 2]
        v3 = cv[:, :, 3]
        scls = jax.nn.sigmoid(cv[:, :, 5:_NCH])
        lastf = is_last.astype(jnp.float32)
        sc2 = sc * sc
        bxy = (_bce(s0, t0) + _bce(s1, t0)) * sc2
        dwh = (v2 - lwt) ** 2 + (v3 - lht) ** 2
        lxy = jnp.sum(bxy * lastf)
        lwh = jnp.sum(sc2 * dwh * lastf) * 0.5
        lcls = jnp.sum(jnp.sum(_bce(scls, unif), axis=2) * lastf)
        l2 = jnp.sum(
            ((s0 - t0) ** 2 + (s1 - t0) ** 2 + sc2 * dwh
             + jnp.sum((scls - unif) ** 2, axis=2)) * lastf
        )
        lane128 = lax.broadcasted_iota(jnp.int32, (1, 128), 1)
        acc = (jnp.where(lane128 == 2, lxy, 0.0)
               + jnp.where(lane128 == 3, lwh, 0.0)
               + jnp.where(lane128 == 4, lcls, 0.0)
               + jnp.where(lane128 == 5, l2, 0.0))
        out_ref[...] = acc

    return pl.pallas_call(
        body,
        out_shape=jax.ShapeDtypeStruct((1, 128), jnp.float32),
    )(cellv, metaf, metai)


def kernel(x, label):
    xr = x.reshape(_B, _A, _NCH, _F * _F)
    xflat = x.reshape(-1)
    metaf, metai, idx = _label_prep(label)
    idx2 = idx.reshape(_NTILES, _PER_TILE)
    gath = _gather_cells(xflat, idx2)
    cellv = gath.reshape(_B, _NL, _CHP)
    pa = _dense_pass(xr, metaf, metai)
    pb = _corrections(cellv, metaf, metai)
    loss_xy = pb[0, 2]
    loss_wh = pb[0, 3]
    loss_obj = pa[0, 0]
    loss_cls = pb[0, 4]
    loss_l2 = pa[0, 1] + pb[0, 5]
    loss = loss_xy + loss_wh + loss_obj + loss_cls
    return (loss, loss_xy, loss_wh, loss_obj, loss_cls, loss_l2)
